# baseline (speedup 1.0000x reference)
import functools

import jax
import jax.numpy as jnp
from jax import lax
from jax.experimental import pallas as pl
from jax.experimental.pallas import tpu as pltpu
from jax.experimental.pallas import tpu_sc as plsc

NTOKEN = 100000
EMB_DIM = 64
BATCH = 4096
HIST = 50
TOT = BATCH * HIST

_info = plsc.get_sparse_core_info()
NC = _info.num_cores
NS = _info.num_subcores
NW = NC * NS
BPW = TOT // NW
CHUNK = 800
NCHUNK = BPW // CHUNK

_mesh = plsc.VectorSubcoreMesh(core_axis_name="c", subcore_axis_name="s")


@functools.partial(
    pl.kernel,
    mesh=_mesh,
    out_type=jax.ShapeDtypeStruct((TOT, EMB_DIM), jnp.float32),
    scratch_types=[
        pltpu.VMEM((CHUNK,), jnp.int32),
        pltpu.VMEM((CHUNK,), jnp.int32),
        pltpu.VMEM((CHUNK, EMB_DIM), jnp.float32),
        pltpu.VMEM((CHUNK, EMB_DIM), jnp.float32),
        pltpu.SemaphoreType.DMA,
        pltpu.SemaphoreType.DMA,
        pltpu.SemaphoreType.DMA,
        pltpu.SemaphoreType.DMA,
    ],
    compiler_params=pltpu.CompilerParams(use_tc_tiling_on_sc=False),
)
def _gather_kernel(idx_hbm, table_hbm, out_hbm, i0, i1, r0, r1, gs0, gs1, os0, os1):
    wid = lax.axis_index("s") * NC + lax.axis_index("c")
    base = wid * BPW
    ib, rb, gs, os_ = [i0, i1], [r0, r1], [gs0, gs1], [os0, os1]

    def off(i):
        return base + i * CHUNK

    gathers = [None] * NCHUNK
    outs = [None] * NCHUNK
    for i in range(min(2, NCHUNK)):
        b = i % 2
        pltpu.sync_copy(idx_hbm.at[pl.ds(off(i), CHUNK)], ib[b])
        gathers[i] = pltpu.async_copy(table_hbm.at[ib[b]], rb[b], gs[b])
    for i in range(NCHUNK):
        b = i % 2
        gathers[i].wait()
        outs[i] = pltpu.async_copy(rb[b], out_hbm.at[pl.ds(off(i), CHUNK)], os_[b])
        if i + 2 < NCHUNK:
            outs[i].wait()
            pltpu.sync_copy(idx_hbm.at[pl.ds(off(i + 2), CHUNK)], ib[b])
            gathers[i + 2] = pltpu.async_copy(table_hbm.at[ib[b]], rb[b], gs[b])
    outs[NCHUNK - 2].wait()
    outs[NCHUNK - 1].wait()


_RB = 32
_GRID = BATCH // _RB


def _reshape_body(rows_ref, out_ref):
    out_ref[...] = rows_ref[...].reshape(_RB, HIST, EMB_DIM)


_reshape_tc = pl.pallas_call(
    _reshape_body,
    grid=(_GRID,),
    in_specs=[pl.BlockSpec((_RB * HIST, EMB_DIM), lambda i: (i, 0))],
    out_specs=pl.BlockSpec((_RB, HIST, EMB_DIM), lambda i: (i, 0, 0)),
    out_shape=jax.ShapeDtypeStruct((BATCH, HIST, EMB_DIM), jnp.float32),
)


def kernel(x, table):
    flat = x.reshape(TOT)
    out = _gather_kernel(flat, table[:NTOKEN])
    return _reshape_tc(out)


# speedup vs baseline: 1.1290x; 1.1290x over previous
import functools

import jax
import jax.numpy as jnp
from jax import lax
from jax.experimental import pallas as pl
from jax.experimental.pallas import tpu as pltpu
from jax.experimental.pallas import tpu_sc as plsc

NTOKEN = 100000
EMB_DIM = 64
BATCH = 4096
HIST = 50
TOT = BATCH * HIST
LANES = 16
NVEC = EMB_DIM // LANES

_info = plsc.get_sparse_core_info()
NC = _info.num_cores
NS = _info.num_subcores
NW = NC * NS

_mesh = plsc.VectorSubcoreMesh(core_axis_name="c", subcore_axis_name="s")

BPW = TOT // NW
CHUNK = 800
NCHUNK = BPW // CHUNK


@functools.partial(
    pl.kernel,
    mesh=_mesh,
    out_type=jax.ShapeDtypeStruct((TOT, EMB_DIM), jnp.float32),
    scratch_types=[
        pltpu.VMEM((CHUNK,), jnp.int32),
        pltpu.VMEM((CHUNK,), jnp.int32),
        pltpu.VMEM((CHUNK, EMB_DIM), jnp.float32),
        pltpu.VMEM((CHUNK, EMB_DIM), jnp.float32),
        pltpu.SemaphoreType.DMA,
        pltpu.SemaphoreType.DMA,
        pltpu.SemaphoreType.DMA,
        pltpu.SemaphoreType.DMA,
    ],
    compiler_params=pltpu.CompilerParams(use_tc_tiling_on_sc=False),
)
def _gather_kernel(idx_hbm, table_hbm, out_hbm, i0, i1, r0, r1, gs0, gs1, os0, os1):
    wid = lax.axis_index("s") * NC + lax.axis_index("c")
    base = wid * BPW
    ib, rb, gs, os_ = [i0, i1], [r0, r1], [gs0, gs1], [os0, os1]

    def off(i):
        return base + i * CHUNK

    gathers = [None] * NCHUNK
    outs = [None] * NCHUNK
    for i in range(min(2, NCHUNK)):
        b = i % 2
        pltpu.sync_copy(idx_hbm.at[pl.ds(off(i), CHUNK)], ib[b])
        gathers[i] = pltpu.async_copy(table_hbm.at[ib[b]], rb[b], gs[b])
    for i in range(NCHUNK):
        b = i % 2
        gathers[i].wait()
        outs[i] = pltpu.async_copy(rb[b], out_hbm.at[pl.ds(off(i), CHUNK)], os_[b])
        if i + 2 < NCHUNK:
            outs[i].wait()
            pltpu.sync_copy(idx_hbm.at[pl.ds(off(i + 2), CHUNK)], ib[b])
            gathers[i + 2] = pltpu.async_copy(table_hbm.at[ib[b]], rb[b], gs[b])
    outs[NCHUNK - 2].wait()
    outs[NCHUNK - 1].wait()


GB2 = 4
GROWS = GB2 * HIST
BPW2 = BATCH // NW
NG = BPW2 // GB2


@functools.partial(
    pl.kernel,
    mesh=_mesh,
    out_type=jax.ShapeDtypeStruct((BATCH, HIST, EMB_DIM), jnp.float32),
    scratch_types=[
        pltpu.VMEM((GROWS, EMB_DIM), jnp.float32),
        pltpu.VMEM((GROWS, EMB_DIM), jnp.float32),
        pltpu.VMEM((GB2, HIST, EMB_DIM), jnp.float32),
        pltpu.VMEM((GB2, HIST, EMB_DIM), jnp.float32),
        pltpu.SemaphoreType.DMA,
        pltpu.SemaphoreType.DMA,
        pltpu.SemaphoreType.DMA,
        pltpu.SemaphoreType.DMA,
    ],
    compiler_params=pltpu.CompilerParams(use_tc_tiling_on_sc=True),
)
def _relayout_kernel(rows_hbm, out_hbm, f0, f1, t0, t1, ls0, ls1, ss0, ss1):
    wid = lax.axis_index("s") * NC + lax.axis_index("c")
    b0 = pl.multiple_of(wid * BPW2, BPW2)
    fb, tb, ls, ss = [f0, f1], [t0, t1], [ls0, ls1], [ss0, ss1]

    def row_off(i):
        return pl.multiple_of((b0 + i * GB2) * HIST, GROWS)

    def load_start(i, b):
        return pltpu.async_copy(rows_hbm.at[pl.ds(row_off(i), GROWS)], fb[b], ls[b])

    def load_wait(i, b):
        pltpu.make_async_copy(rows_hbm.at[pl.ds(row_off(i), GROWS)], fb[b], ls[b]).wait()

    def store_start(i, b):
        bo = pl.multiple_of(b0 + i * GB2, GB2)
        return pltpu.async_copy(tb[b], out_hbm.at[pl.ds(bo, GB2)], ss[b])

    def store_wait(b):
        pltpu.make_async_copy(tb[b], out_hbm.at[pl.ds(b0, GB2)], ss[b]).wait()

    def veccopy(b):
        for j in range(GB2):
            for l in range(HIST):
                for k in range(NVEC):
                    sl = pl.ds(k * LANES, LANES)
                    tb[b][j, l, sl] = fb[b][j * HIST + l, sl]

    load_start(0, 0)
    load_start(1, 1)
    for i in range(2):
        load_wait(i, i)
        veccopy(i)
        store_start(i, i)
        load_start(i + 2, i)

    def body(g, carry):
        for sub in range(2):
            i = 2 * g + sub
            load_wait(i, sub)
            store_wait(sub)
            veccopy(sub)
            store_start(i, sub)

            @pl.when(g < NG // 2 - 1)
            def _():
                load_start(i + 2, sub)
        return carry

    lax.fori_loop(1, NG // 2, body, 0)
    store_wait(0)
    store_wait(1)


def kernel(x, table):
    flat = x.reshape(TOT)
    rows = _gather_kernel(flat, table)
    return _relayout_kernel(rows)


# speedup vs baseline: 2.0997x; 1.8598x over previous
import functools

import jax
import jax.numpy as jnp
from jax import lax
from jax.experimental import pallas as pl
from jax.experimental.pallas import tpu as pltpu
from jax.experimental.pallas import tpu_sc as plsc

NTOKEN = 100000
EMB_DIM = 64
BATCH = 4096
HIST = 50
LANES = 16
NGRP = BATCH // LANES

_info = plsc.get_sparse_core_info()
NC = _info.num_cores
NS = _info.num_subcores
NW = NC * NS
NPASS = EMB_DIM // NW

_mesh = plsc.VectorSubcoreMesh(core_axis_name="c", subcore_axis_name="s")


@functools.partial(
    pl.kernel,
    mesh=_mesh,
    out_type=jax.ShapeDtypeStruct((HIST, EMB_DIM, BATCH), jnp.float32),
    scratch_types=[
        pltpu.VMEM((NTOKEN + 1,), jnp.float32),
        pltpu.VMEM((BATCH,), jnp.int32),
        pltpu.VMEM((BATCH,), jnp.int32),
        pltpu.VMEM((BATCH,), jnp.float32),
        pltpu.VMEM((BATCH,), jnp.float32),
        pltpu.SemaphoreType.DMA,
        pltpu.SemaphoreType.DMA,
        pltpu.SemaphoreType.DMA,
        pltpu.SemaphoreType.DMA,
    ],
    compiler_params=pltpu.CompilerParams(use_tc_tiling_on_sc=True,
                                         needs_layout_passes=False),
)
def _tgather_kernel(tableT_hbm, xT_hbm, out_hbm, row_v, x0, x1, s0, s1,
                    lx0, lx1, ss0, ss1):
    wid = lax.axis_index("s") * NC + lax.axis_index("c")
    xv, sv, lx, ss = [x0, x1], [s0, s1], [lx0, lx1], [ss0, ss1]

    def gather_slab(sub):
        for i in range(NGRP):
            sl = pl.ds(i * LANES, LANES)
            sv[sub][sl] = plsc.load_gather(row_v, [xv[sub][sl]])

    for p in range(NPASS):
        e = wid + NW * p
        pltpu.async_copy(xT_hbm.at[0], xv[0], lx[0])
        pltpu.async_copy(xT_hbm.at[1], xv[1], lx[1])
        pltpu.sync_copy(tableT_hbm.at[e], row_v)

        def pair(g, carry):
            for sub in range(2):
                l = 2 * g + sub
                pltpu.make_async_copy(xT_hbm.at[l], xv[sub], lx[sub]).wait()

                @pl.when(g >= 1)
                def _wait_store():
                    pltpu.make_async_copy(sv[sub], out_hbm.at[l, e], ss[sub]).wait()

                gather_slab(sub)
                pltpu.async_copy(sv[sub], out_hbm.at[l, e], ss[sub])

                @pl.when(g < HIST // 2 - 1)
                def _prefetch():
                    pltpu.async_copy(xT_hbm.at[l + 2], xv[sub], lx[sub])
            return carry

        lax.fori_loop(0, HIST // 2, pair, 0)
        for sub in range(2):
            pltpu.make_async_copy(sv[sub], out_hbm.at[0, e], ss[sub]).wait()


def kernel(x, table):
    out_t = _tgather_kernel(table.T, x.T.astype(jnp.int32))
    return out_t.transpose(2, 0, 1)


# speedup vs baseline: 3.3364x; 1.5890x over previous
import functools

import jax
import jax.numpy as jnp
from jax import lax
from jax.experimental import pallas as pl
from jax.experimental.pallas import tpu as pltpu
from jax.experimental.pallas import tpu_sc as plsc

NTOKEN = 100000
EMB_DIM = 64
BATCH = 4096
HIST = 50
LANES = 16
NGRP = BATCH // LANES

_info = plsc.get_sparse_core_info()
NC = _info.num_cores
NS = _info.num_subcores
NW = NC * NS
NPASS = EMB_DIM // NW

_mesh = plsc.VectorSubcoreMesh(core_axis_name="c", subcore_axis_name="s")


@functools.partial(
    pl.kernel,
    mesh=_mesh,
    out_type=jax.ShapeDtypeStruct((HIST, EMB_DIM, BATCH), jnp.float32),
    scratch_types=[
        pltpu.VMEM((NTOKEN + 1,), jnp.float32),
        pltpu.VMEM((BATCH,), jnp.int32),
        pltpu.VMEM((BATCH,), jnp.int32),
        pltpu.VMEM((BATCH,), jnp.float32),
        pltpu.VMEM((BATCH,), jnp.float32),
        pltpu.SemaphoreType.DMA,
        pltpu.SemaphoreType.DMA,
        pltpu.SemaphoreType.DMA,
        pltpu.SemaphoreType.DMA,
    ],
    compiler_params=pltpu.CompilerParams(use_tc_tiling_on_sc=True,
                                         needs_layout_passes=False),
)
def _tgather_kernel(tableT_hbm, xT_hbm, out_hbm, row_v, x0, x1, s0, s1,
                    lx0, lx1, ss0, ss1):
    wid = lax.axis_index("s") * NC + lax.axis_index("c")
    xv, sv, lx, ss = [x0, x1], [s0, s1], [lx0, lx1], [ss0, ss1]

    def gather_slab(sub):
        W = 6
        idxs = [None] * NGRP
        vals = [None] * NGRP
        for i in range(NGRP + 2 * W):
            if i < NGRP:
                idxs[i] = xv[sub][pl.ds(i * LANES, LANES)]
            j = i - W
            if 0 <= j < NGRP:
                vals[j] = plsc.load_gather(row_v, [idxs[j]])
                idxs[j] = None
            k = i - 2 * W
            if 0 <= k < NGRP:
                sv[sub][pl.ds(k * LANES, LANES)] = vals[k]
                vals[k] = None

    for p in range(NPASS):
        e = wid + NW * p
        pltpu.async_copy(xT_hbm.at[0], xv[0], lx[0])
        pltpu.async_copy(xT_hbm.at[1], xv[1], lx[1])
        pltpu.sync_copy(tableT_hbm.at[e], row_v)

        def pair(g, carry):
            for sub in range(2):
                l = 2 * g + sub
                pltpu.make_async_copy(xT_hbm.at[l], xv[sub], lx[sub]).wait()

                @pl.when(g >= 1)
                def _wait_store():
                    pltpu.make_async_copy(sv[sub], out_hbm.at[l, e], ss[sub]).wait()

                gather_slab(sub)
                pltpu.async_copy(sv[sub], out_hbm.at[l, e], ss[sub])

                @pl.when(g < HIST // 2 - 1)
                def _prefetch():
                    pltpu.async_copy(xT_hbm.at[l + 2], xv[sub], lx[sub])
            return carry

        lax.fori_loop(0, HIST // 2, pair, 0)
        for sub in range(2):
            pltpu.make_async_copy(sv[sub], out_hbm.at[0, e], ss[sub]).wait()


def kernel(x, table):
    out_t = _tgather_kernel(table.T, x.T.astype(jnp.int32))
    return out_t.transpose(2, 0, 1)
